# baseline (speedup 1.0000x reference)
import jax
import jax.numpy as jnp
from jax import lax
from jax.experimental import pallas as pl
from jax.experimental.pallas import tpu as pltpu
from jax.experimental.pallas import tpu_sc as plsc

B = 256
N = 1024
D = 192
NP1 = N + 1
NC, NS = 2, 16
NW = NC * NS
RW = N // NW
GR = RW + 8
NB = 2
NCHUNK = B // NB
LANES = 16
NL = D // LANES


def _body(patch_hbm, pos_hbm, out_hbm,
          pos_v, post_v, ibuf0, ibuf1, obuf0, obuf1, tbuf0, tbuf1,
          gsem0, gsem1, ssem0, ssem1, tsem0, tsem1):
    cid = lax.axis_index("c")
    sid = lax.axis_index("s")
    wid = sid * NC + cid
    r0 = wid * RW
    is_first = wid == 0
    not_first = wid != 0
    is_last = wid == NW - 1

    pltpu.sync_copy(pos_hbm.at[pl.ds(r0, RW), :], pos_v)
    pltpu.sync_copy(pos_hbm.at[pl.ds(N, 1), :], post_v)

    ibufs = (ibuf0, ibuf1)
    obufs = (obuf0, obuf1)
    tbufs = (tbuf0, tbuf1)
    gsems = (gsem0, gsem1)
    ssems = (ssem0, ssem1)
    tsems = (tsem0, tsem1)

    @pl.when(is_first)
    def _():
        zero = jnp.zeros((LANES,), jnp.float32)
        for slot in range(2):
            for b in range(NB):
                for l in range(NL):
                    ibufs[slot][b, 7, pl.ds(l * LANES, LANES)] = zero

    def g_desc_main(c, slot):
        return pltpu.make_async_copy(
            patch_hbm.at[pl.ds(c * NB, NB), pl.ds(r0 - 8, GR), :],
            ibufs[slot], gsems[slot])

    def g_desc_first(c, slot):
        return pltpu.make_async_copy(
            patch_hbm.at[pl.ds(c * NB, NB), pl.ds(0, RW), :],
            ibufs[slot].at[:, pl.ds(8, RW), :], gsems[slot])

    def gather_start(c, slot):
        @pl.when(is_first)
        def _():
            g_desc_first(c, slot).start()

        @pl.when(not_first)
        def _():
            g_desc_main(c, slot).start()

    def gather_wait(c, slot):
        @pl.when(is_first)
        def _():
            g_desc_first(c, slot).wait()

        @pl.when(not_first)
        def _():
            g_desc_main(c, slot).wait()

    def s_desc(c, slot):
        return pltpu.make_async_copy(
            obufs[slot],
            out_hbm.at[pl.ds(c * NB, NB), pl.ds(r0, RW), :],
            ssems[slot])

    def t_desc(c, slot):
        return pltpu.make_async_copy(
            tbufs[slot],
            out_hbm.at[pl.ds(c * NB, NB), pl.ds(N, 1), :],
            tsems[slot])

    def compute(slot):
        ib, ob = ibufs[slot], obufs[slot]

        def jbody(j, carry):
            for l in range(NL):
                preg = pos_v[j, pl.ds(l * LANES, LANES)]
                for b in range(NB):
                    ob[b, j, pl.ds(l * LANES, LANES)] = (
                        ib[b, j + 7, pl.ds(l * LANES, LANES)] + preg)
            return carry

        lax.fori_loop(0, RW, jbody, 0)

        @pl.when(is_last)
        def _():
            tb = tbufs[slot]
            for l in range(NL):
                treg = post_v[0, pl.ds(l * LANES, LANES)]
                for b in range(NB):
                    tb[b, 0, pl.ds(l * LANES, LANES)] = (
                        ib[b, RW + 7, pl.ds(l * LANES, LANES)] + treg)

    gather_start(0, 0)
    gather_start(1, 1)

    def outer(i, carry):
        for slot in range(2):
            c = 2 * i + slot
            gather_wait(c, slot)

            @pl.when(c >= 2)
            def _():
                s_desc(c - 2, slot).wait()

                @pl.when(is_last)
                def _():
                    t_desc(c - 2, slot).wait()

            compute(slot)
            s_desc(c, slot).start()

            @pl.when(is_last)
            def _():
                t_desc(c, slot).start()

            @pl.when(c + 2 < NCHUNK)
            def _():
                gather_start(c + 2, slot)
        return carry

    lax.fori_loop(0, NCHUNK // 2, outer, 0)

    for slot in range(2):
        s_desc(NCHUNK - 2 + slot, slot).wait()

        @pl.when(is_last)
        def _():
            t_desc(NCHUNK - 2 + slot, slot).wait()


def kernel(patch, pos_table):
    mesh = plsc.VectorSubcoreMesh(core_axis_name="c", subcore_axis_name="s")
    f = pl.kernel(
        _body,
        out_type=jax.ShapeDtypeStruct((B, NP1, D), jnp.float32),
        mesh=mesh,
        scratch_types=[
            pltpu.VMEM((RW, D), jnp.float32),
            pltpu.VMEM((1, D), jnp.float32),
            pltpu.VMEM((NB, GR, D), jnp.float32),
            pltpu.VMEM((NB, GR, D), jnp.float32),
            pltpu.VMEM((NB, RW, D), jnp.float32),
            pltpu.VMEM((NB, RW, D), jnp.float32),
            pltpu.VMEM((NB, 1, D), jnp.float32),
            pltpu.VMEM((NB, 1, D), jnp.float32),
            pltpu.SemaphoreType.DMA,
            pltpu.SemaphoreType.DMA,
            pltpu.SemaphoreType.DMA,
            pltpu.SemaphoreType.DMA,
            pltpu.SemaphoreType.DMA,
            pltpu.SemaphoreType.DMA,
        ],
    )
    return f(patch, pos_table)
